# Initial kernel scaffold; baseline (speedup 1.0000x reference)
import functools

import jax
import jax.numpy as jnp
from jax import lax
from jax.experimental import pallas as pl
from jax.experimental.pallas import tpu as pltpu
from jax.experimental.pallas import tpu_sc as plsc

B, N, NP, K = 4, 2048, 1024, 16
C1, C2 = 128, 256
NR, NC = 16, 128


def _fps_body(xyz_ref, nxr_ref):
    x = xyz_ref[0, 0]
    y = xyz_ref[0, 1]
    z = xyz_ref[0, 2]
    iota2 = (lax.broadcasted_iota(jnp.int32, (NR, NC), 0) * NC
             + lax.broadcasted_iota(jnp.int32, (NR, NC), 1))
    bigi = jnp.int32(2 ** 30)

    def body(i, carry):
        dist, far = carry
        sel = iota2 == far
        cx = jnp.sum(jnp.where(sel, x, 0.0))
        cy = jnp.sum(jnp.where(sel, y, 0.0))
        cz = jnp.sum(jnp.where(sel, z, 0.0))
        nxr_ref[pl.ds(i, 1), :] = jnp.concatenate(
            [cx.reshape(1, 1), cy.reshape(1, 1), cz.reshape(1, 1)], axis=1)
        d = (x - cx) ** 2 + (y - cy) ** 2 + (z - cz) ** 2
        dist = jnp.minimum(dist, d)
        m = jnp.max(dist)
        far = jnp.min(jnp.where(dist == m, iota2, bigi))
        return dist, far

    dist0 = jnp.full((NR, NC), 1e10, dtype=jnp.float32)
    lax.fori_loop(0, NP, body, (dist0, jnp.int32(0)))


def _run_fps(xyz):
    xyz4 = xyz.reshape(B, 3, NR, NC)
    return pl.pallas_call(
        _fps_body,
        grid=(B,),
        in_specs=[pl.BlockSpec((1, 3, NR, NC), lambda b: (b, 0, 0, 0))],
        out_specs=pl.BlockSpec((1, NP, 3), lambda b: (b, 0, 0)),
        out_shape=jax.ShapeDtypeStruct((B, NP, 3), jnp.float32),
    )(xyz4)


MT = 128
NT_ROWS = NP // MT


def _knn_body(nxr_ref, xyz_ref, idx_ref, cnt_ref):
    b = pl.program_id(0)
    t = pl.program_id(1)
    nx = nxr_ref[0]
    x = xyz_ref[0]
    sq_pts = jnp.sum(x * x, axis=0)
    sq_new = jnp.sum(nx * nx, axis=1)
    dot = jnp.dot(nx, x, preferred_element_type=jnp.float32)
    sqd = (sq_new[:, None] + sq_pts[None, :]) - 2.0 * dot

    lane_idx = lax.broadcasted_iota(jnp.int32, (MT, N), 1)
    col16 = lax.broadcasted_iota(jnp.int32, (MT, K), 1)
    bigi = jnp.int32(2 ** 30)
    inf = jnp.float32(jnp.inf)
    work = sqd
    idx_block = jnp.zeros((MT, K), dtype=jnp.int32)
    for j in range(K):
        m = jnp.min(work, axis=1, keepdims=True)
        cand = jnp.where(work == m, lane_idx, bigi)
        sel = jnp.min(cand, axis=1, keepdims=True)
        idx_block = jnp.where(col16 == j, jnp.broadcast_to(sel, (MT, K)),
                              idx_block)
        work = jnp.where(lane_idx == sel, inf, work)

    idx_ref[0] = idx_block + b * N
    cnt_tile = jnp.sum((work == inf).astype(jnp.float32), axis=0)

    @pl.when(t == 0)
    def _():
        cnt_ref[0, 0, :] = jnp.zeros((N,), jnp.float32)

    cnt_ref[0, 0, :] += cnt_tile


def _run_knn(nxr, xyz):
    return pl.pallas_call(
        _knn_body,
        grid=(B, NT_ROWS),
        in_specs=[
            pl.BlockSpec((1, MT, 3), lambda b, t: (b, t, 0)),
            pl.BlockSpec((1, 3, N), lambda b, t: (b, 0, 0)),
        ],
        out_specs=[
            pl.BlockSpec((1, MT, K), lambda b, t: (b, t, 0)),
            pl.BlockSpec((1, 1, N), lambda b, t: (b, 0, 0)),
        ],
        out_shape=[
            jax.ShapeDtypeStruct((B, NP, K), jnp.int32),
            jax.ShapeDtypeStruct((B, 1, N), jnp.float32),
        ],
    )(nxr, xyz)


def _mlp_body(p_ref, cnt_ref, w1t_ref, b1_ref, g1_ref, be1_ref,
              w2t_ref, b2_ref, g2_ref, be2_ref, out_ref, h1s_ref, h2s_ref):
    ntot = jnp.float32(B * NP * K)
    hi = jax.lax.Precision.HIGHEST
    s1 = jnp.zeros((1, C1), jnp.float32)
    q1 = jnp.zeros((1, C1), jnp.float32)
    for b in range(B):
        h = jnp.dot(p_ref[b], w1t_ref[...],
                    preferred_element_type=jnp.float32) + b1_ref[...]
        h1s_ref[b] = h
        c = cnt_ref[b]
        s1 += jnp.dot(c, h, preferred_element_type=jnp.float32, precision=hi)
        q1 += jnp.dot(c, h * h, preferred_element_type=jnp.float32,
                      precision=hi)
    m1 = s1 / ntot
    v1 = q1 / ntot - m1 * m1
    sc1 = g1_ref[...] / jnp.sqrt(v1 + 1e-5)
    sh1 = be1_ref[...] - m1 * sc1

    s2 = jnp.zeros((1, C2), jnp.float32)
    q2 = jnp.zeros((1, C2), jnp.float32)
    for b in range(B):
        h1 = jnp.maximum(h1s_ref[b] * sc1 + sh1, 0.0)
        h2 = jnp.dot(h1, w2t_ref[...],
                     preferred_element_type=jnp.float32) + b2_ref[...]
        h2s_ref[b] = h2
        c = cnt_ref[b]
        s2 += jnp.dot(c, h2, preferred_element_type=jnp.float32, precision=hi)
        q2 += jnp.dot(c, h2 * h2, preferred_element_type=jnp.float32,
                      precision=hi)
    m2 = s2 / ntot
    v2 = q2 / ntot - m2 * m2
    sc2 = g2_ref[...] / jnp.sqrt(v2 + 1e-5)
    sh2 = be2_ref[...] - m2 * sc2
    for b in range(B):
        out_ref[b] = jnp.maximum(h2s_ref[b] * sc2 + sh2, 0.0)


def _run_mlp(p_pm, counts, W1, b1, gamma1, beta1, W2, b2, gamma2, beta2):
    shapes = [(B, N, C1), (B, 1, N), (C1, C1), (1, C1), (1, C1), (1, C1),
              (C1, C2), (1, C2), (1, C2), (1, C2)]
    return pl.pallas_call(
        _mlp_body,
        grid=(1,),
        in_specs=[pl.BlockSpec(s, lambda _, _s=s: tuple(0 for _ in _s))
                  for s in shapes],
        out_specs=pl.BlockSpec((B, N, C2), lambda _: (0, 0, 0)),
        out_shape=jax.ShapeDtypeStruct((B, N, C2), jnp.float32),
        scratch_shapes=[
            pltpu.VMEM((B, N, C1), jnp.float32),
            pltpu.VMEM((B, N, C2), jnp.float32),
        ],
    )(p_pm, counts, W1.T, b1.reshape(1, C1), gamma1.reshape(1, C1),
      beta1.reshape(1, C1), W2.T, b2.reshape(1, C2), gamma2.reshape(1, C2),
      beta2.reshape(1, C2))


NWORK = 32
RPW = (B * NP) // NWORK
G = 4


def _pool_body(h2_hbm, idx_hbm, out_hbm, idx_v, bufs, out_v, sem):
    wid = lax.axis_index("s") * 2 + lax.axis_index("c")
    base = wid * RPW
    pltpu.sync_copy(idx_hbm.at[pl.ds(base, RPW)], idx_v)

    def chunk(ci, _):
        r0 = ci * G
        handles = []
        for j in range(G):
            handles.append(pltpu.async_copy(
                h2_hbm.at[idx_v.at[r0 + j]], bufs.at[j], sem))
        for h in handles:
            h.wait()
        for j in range(G):
            for c in range(C2 // 16):
                acc = bufs[j, 0, pl.ds(c * 16, 16)]
                for r in range(1, K):
                    acc = jnp.maximum(acc, bufs[j, r, pl.ds(c * 16, 16)])
                out_v[r0 + j, pl.ds(c * 16, 16)] = acc
        return 0

    lax.fori_loop(0, RPW // G, chunk, 0)
    pltpu.sync_copy(out_v, out_hbm.at[pl.ds(base, RPW)])


@functools.partial(
    pl.kernel,
    mesh=plsc.VectorSubcoreMesh(core_axis_name="c", subcore_axis_name="s"),
    out_type=jax.ShapeDtypeStruct((B * NP, C2), jnp.float32),
    scratch_types=[
        pltpu.VMEM((RPW, K), jnp.int32),
        pltpu.VMEM((G, K, C2), jnp.float32),
        pltpu.VMEM((RPW, C2), jnp.float32),
        pltpu.SemaphoreType.DMA,
    ],
)
def _pool_kernel(h2_hbm, idx_hbm, out_hbm, idx_v, bufs, out_v, sem):
    _pool_body(h2_hbm, idx_hbm, out_hbm, idx_v, bufs, out_v, sem)


def kernel(xyz, points, W1, b1, gamma1, beta1, W2, b2, gamma2, beta2):
    nxr = _run_fps(xyz)
    idxg, counts = _run_knn(nxr, xyz)
    p_pm = jnp.transpose(points, (0, 2, 1))
    h2 = _run_mlp(p_pm, counts, W1, b1, gamma1, beta1,
                  W2, b2, gamma2, beta2)
    pooled = _pool_kernel(h2.reshape(B * N, C2),
                          idxg.reshape(B * NP, K))
    new_xyz = jnp.transpose(nxr, (0, 2, 1))
    return new_xyz, jnp.transpose(pooled.reshape(B, NP, C2), (0, 2, 1))


# speedup vs baseline: 3.6059x; 3.6059x over previous
import functools

import jax
import jax.numpy as jnp
from jax import lax
from jax.experimental import pallas as pl
from jax.experimental.pallas import tpu as pltpu
from jax.experimental.pallas import tpu_sc as plsc

B, N, NP, K = 4, 2048, 1024, 16
C1, C2 = 128, 256
NR, NC = 16, 128


def _fps_body(xyz_ref, nxr_ref):
    x = xyz_ref[0, 0]
    y = xyz_ref[0, 1]
    z = xyz_ref[0, 2]
    iota2 = (lax.broadcasted_iota(jnp.int32, (NR, NC), 0) * NC
             + lax.broadcasted_iota(jnp.int32, (NR, NC), 1))
    bigi = jnp.int32(2 ** 30)

    def body(i, carry):
        dist, far = carry
        sel = iota2 == far
        cx = jnp.sum(jnp.where(sel, x, 0.0))
        cy = jnp.sum(jnp.where(sel, y, 0.0))
        cz = jnp.sum(jnp.where(sel, z, 0.0))
        nxr_ref[0, pl.ds(i, 1), :] = jnp.concatenate(
            [cx.reshape(1, 1), cy.reshape(1, 1), cz.reshape(1, 1)], axis=1)
        d = (x - cx) ** 2 + (y - cy) ** 2 + (z - cz) ** 2
        dist = jnp.minimum(dist, d)
        m = jnp.max(dist)
        far = jnp.min(jnp.where(dist == m, iota2, bigi))
        return dist, far

    dist0 = jnp.full((NR, NC), 1e10, dtype=jnp.float32)
    lax.fori_loop(0, NP, body, (dist0, jnp.int32(0)))


def _run_fps(xyz):
    xyz4 = xyz.reshape(B, 3, NR, NC)
    return pl.pallas_call(
        _fps_body,
        grid=(B,),
        in_specs=[pl.BlockSpec((1, 3, NR, NC), lambda b: (b, 0, 0, 0))],
        out_specs=pl.BlockSpec((1, NP, 3), lambda b: (b, 0, 0)),
        out_shape=jax.ShapeDtypeStruct((B, NP, 3), jnp.float32),
    )(xyz4)


MT = 128
NT_ROWS = NP // MT


def _knn_body(nxr_ref, xyz_ref, idx_ref, cnt_ref):
    b = pl.program_id(0)
    t = pl.program_id(1)
    nx = nxr_ref[0]
    x = xyz_ref[0]
    sq_pts = jnp.sum(x * x, axis=0)
    sq_new = jnp.sum(nx * nx, axis=1)
    dot = jnp.dot(nx, x, preferred_element_type=jnp.float32)
    sqd = (sq_new[:, None] + sq_pts[None, :]) - 2.0 * dot

    lane_idx = lax.broadcasted_iota(jnp.int32, (MT, N), 1)
    col16 = lax.broadcasted_iota(jnp.int32, (MT, K), 1)
    bigi = jnp.int32(2 ** 30)
    inf = jnp.float32(jnp.inf)
    work = sqd
    idx_block = jnp.zeros((MT, K), dtype=jnp.int32)
    for j in range(K):
        m = jnp.min(work, axis=1, keepdims=True)
        cand = jnp.where(work == m, lane_idx, bigi)
        sel = jnp.min(cand, axis=1, keepdims=True)
        idx_block = jnp.where(col16 == j, jnp.broadcast_to(sel, (MT, K)),
                              idx_block)
        work = jnp.where(lane_idx == sel, inf, work)

    idx_ref[0] = idx_block + b * N
    cnt_tile = jnp.sum((work == inf).astype(jnp.float32), axis=0)

    @pl.when(t == 0)
    def _():
        cnt_ref[0, 0, :] = jnp.zeros((N,), jnp.float32)

    cnt_ref[0, 0, :] += cnt_tile


def _run_knn(nxr, xyz):
    return pl.pallas_call(
        _knn_body,
        grid=(B, NT_ROWS),
        in_specs=[
            pl.BlockSpec((1, MT, 3), lambda b, t: (b, t, 0)),
            pl.BlockSpec((1, 3, N), lambda b, t: (b, 0, 0)),
        ],
        out_specs=[
            pl.BlockSpec((1, MT, K), lambda b, t: (b, t, 0)),
            pl.BlockSpec((1, 1, N), lambda b, t: (b, 0, 0)),
        ],
        out_shape=[
            jax.ShapeDtypeStruct((B, NP, K), jnp.int32),
            jax.ShapeDtypeStruct((B, 1, N), jnp.float32),
        ],
    )(nxr, xyz)


def _mlp_body(p_ref, cnt_ref, w1t_ref, b1_ref, g1_ref, be1_ref,
              w2t_ref, b2_ref, g2_ref, be2_ref, out_ref, h1s_ref, h2s_ref):
    ntot = jnp.float32(B * NP * K)
    hi = jax.lax.Precision.HIGHEST
    s1 = jnp.zeros((1, C1), jnp.float32)
    q1 = jnp.zeros((1, C1), jnp.float32)
    for b in range(B):
        h = jnp.dot(p_ref[b], w1t_ref[...],
                    preferred_element_type=jnp.float32) + b1_ref[...]
        h1s_ref[b] = h
        c = cnt_ref[b]
        s1 += jnp.dot(c, h, preferred_element_type=jnp.float32, precision=hi)
        q1 += jnp.dot(c, h * h, preferred_element_type=jnp.float32,
                      precision=hi)
    m1 = s1 / ntot
    v1 = q1 / ntot - m1 * m1
    sc1 = g1_ref[...] / jnp.sqrt(v1 + 1e-5)
    sh1 = be1_ref[...] - m1 * sc1

    s2 = jnp.zeros((1, C2), jnp.float32)
    q2 = jnp.zeros((1, C2), jnp.float32)
    for b in range(B):
        h1 = jnp.maximum(h1s_ref[b] * sc1 + sh1, 0.0)
        h2 = jnp.dot(h1, w2t_ref[...],
                     preferred_element_type=jnp.float32) + b2_ref[...]
        h2s_ref[b] = h2
        c = cnt_ref[b]
        s2 += jnp.dot(c, h2, preferred_element_type=jnp.float32, precision=hi)
        q2 += jnp.dot(c, h2 * h2, preferred_element_type=jnp.float32,
                      precision=hi)
    m2 = s2 / ntot
    v2 = q2 / ntot - m2 * m2
    sc2 = g2_ref[...] / jnp.sqrt(v2 + 1e-5)
    sh2 = be2_ref[...] - m2 * sc2
    for b in range(B):
        out_ref[b] = jnp.maximum(h2s_ref[b] * sc2 + sh2, 0.0)


def _run_mlp(p_pm, counts, W1, b1, gamma1, beta1, W2, b2, gamma2, beta2):
    shapes = [(B, N, C1), (B, 1, N), (C1, C1), (1, C1), (1, C1), (1, C1),
              (C1, C2), (1, C2), (1, C2), (1, C2)]
    return pl.pallas_call(
        _mlp_body,
        grid=(1,),
        in_specs=[pl.BlockSpec(s, lambda _, _s=s: tuple(0 for _ in _s))
                  for s in shapes],
        out_specs=pl.BlockSpec((B, N, C2), lambda _: (0, 0, 0)),
        out_shape=jax.ShapeDtypeStruct((B, N, C2), jnp.float32),
        scratch_shapes=[
            pltpu.VMEM((B, N, C1), jnp.float32),
            pltpu.VMEM((B, N, C2), jnp.float32),
        ],
    )(p_pm, counts, W1.T, b1.reshape(1, C1), gamma1.reshape(1, C1),
      beta1.reshape(1, C1), W2.T, b2.reshape(1, C2), gamma2.reshape(1, C2),
      beta2.reshape(1, C2))


NWORK = 32
RPW = (B * NP) // NWORK
G = 4


def _pool_body(h2_hbm, idx_hbm, out_hbm, idx_v, bufs, out_v, sem):
    wid = lax.axis_index("s") * 2 + lax.axis_index("c")
    base = wid * RPW
    pltpu.sync_copy(idx_hbm.at[pl.ds(base, RPW)], idx_v)

    def chunk(ci, _):
        r0 = ci * G
        handles = []
        for j in range(G):
            handles.append(pltpu.async_copy(
                h2_hbm.at[idx_v.at[r0 + j]], bufs.at[j], sem))
        for h in handles:
            h.wait()
        for j in range(G):
            for c in range(C2 // 16):
                acc = bufs[j, 0, pl.ds(c * 16, 16)]
                for r in range(1, K):
                    acc = jnp.maximum(acc, bufs[j, r, pl.ds(c * 16, 16)])
                out_v[r0 + j, pl.ds(c * 16, 16)] = acc
        return 0

    lax.fori_loop(0, RPW // G, chunk, 0)
    pltpu.sync_copy(out_v, out_hbm.at[pl.ds(base, RPW)])


@functools.cache
def _make_pool_kernel():
    return pl.kernel(
        _pool_body,
        mesh=plsc.VectorSubcoreMesh(core_axis_name="c", subcore_axis_name="s"),
        out_type=jax.ShapeDtypeStruct((B * NP, C2), jnp.float32),
        scratch_types=[
            pltpu.VMEM((RPW, K), jnp.int32),
            pltpu.VMEM((G, K, C2), jnp.float32),
            pltpu.VMEM((RPW, C2), jnp.float32),
            pltpu.SemaphoreType.DMA,
        ],
    )


def kernel(xyz, points, W1, b1, gamma1, beta1, W2, b2, gamma2, beta2):
    nxr = _run_fps(xyz)
    idxg, counts = _run_knn(nxr, xyz)
    p_pm = jnp.transpose(points, (0, 2, 1))
    h2 = _run_mlp(p_pm, counts, W1, b1, gamma1, beta1,
                  W2, b2, gamma2, beta2)
    pooled = _make_pool_kernel()(h2.reshape(B * N, C2),
                                 idxg.reshape(B * NP, K))
    new_xyz = jnp.transpose(nxr, (0, 2, 1))
    return new_xyz, jnp.transpose(pooled.reshape(B, NP, C2), (0, 2, 1))


# speedup vs baseline: 10.8687x; 3.0142x over previous
import functools

import jax
import jax.numpy as jnp
from jax import lax
from jax.experimental import pallas as pl
from jax.experimental.pallas import tpu as pltpu
from jax.experimental.pallas import tpu_sc as plsc

B, N, NP, K = 4, 2048, 1024, 16
C1, C2 = 128, 256
NR, NC = 16, 128


def _fps_body(xyz_ref, nxr_ref):
    x = xyz_ref[:, 0]
    y = xyz_ref[:, 1]
    z = xyz_ref[:, 2]
    iota2 = (lax.broadcasted_iota(jnp.int32, (B, NR, NC), 1) * NC
             + lax.broadcasted_iota(jnp.int32, (B, NR, NC), 2))
    bigi = jnp.int32(2 ** 30)

    def body(i, carry):
        dist, far = carry
        sel = iota2 == far
        cx = jnp.sum(jnp.where(sel, x, 0.0), axis=(1, 2), keepdims=True)
        cy = jnp.sum(jnp.where(sel, y, 0.0), axis=(1, 2), keepdims=True)
        cz = jnp.sum(jnp.where(sel, z, 0.0), axis=(1, 2), keepdims=True)
        nxr_ref[:, pl.ds(i, 1), :] = jnp.concatenate(
            [cx[:, :, 0], cy[:, :, 0], cz[:, :, 0]], axis=1)[:, None, :]
        d = (x - cx) ** 2 + (y - cy) ** 2 + (z - cz) ** 2
        dist = jnp.minimum(dist, d)
        m = jnp.max(dist, axis=(1, 2), keepdims=True)
        far = jnp.min(jnp.where(dist == m, iota2, bigi), axis=(1, 2),
                      keepdims=True)
        return dist, far

    dist0 = jnp.full((B, NR, NC), 1e10, dtype=jnp.float32)
    far0 = jnp.zeros((B, 1, 1), dtype=jnp.int32)
    lax.fori_loop(0, NP, body, (dist0, far0))


def _run_fps(xyz):
    xyz4 = xyz.reshape(B, 3, NR, NC)
    return pl.pallas_call(
        _fps_body,
        grid=(1,),
        in_specs=[pl.BlockSpec((B, 3, NR, NC), lambda _: (0, 0, 0, 0))],
        out_specs=pl.BlockSpec((B, NP, 3), lambda _: (0, 0, 0)),
        out_shape=jax.ShapeDtypeStruct((B, NP, 3), jnp.float32),
    )(xyz4)


MT = 128
NT_ROWS = NP // MT


def _knn_body(nxr_ref, xyz_ref, idx_ref, cnt_ref):
    b = pl.program_id(0)
    t = pl.program_id(1)
    nx = nxr_ref[0]
    x = xyz_ref[0]
    sq_pts = jnp.sum(x * x, axis=0)
    sq_new = jnp.sum(nx * nx, axis=1)
    dot = jnp.dot(nx, x, preferred_element_type=jnp.float32)
    sqd = (sq_new[:, None] + sq_pts[None, :]) - 2.0 * dot

    lane_idx = lax.broadcasted_iota(jnp.int32, (MT, N), 1)
    col16 = lax.broadcasted_iota(jnp.int32, (MT, K), 1)
    bigi = jnp.int32(2 ** 30)
    inf = jnp.float32(jnp.inf)
    work = sqd
    idx_block = jnp.zeros((MT, K), dtype=jnp.int32)
    for j in range(K):
        m = jnp.min(work, axis=1, keepdims=True)
        cand = jnp.where(work == m, lane_idx, bigi)
        sel = jnp.min(cand, axis=1, keepdims=True)
        idx_block = jnp.where(col16 == j, jnp.broadcast_to(sel, (MT, K)),
                              idx_block)
        work = jnp.where(lane_idx == sel, inf, work)

    idx_ref[0] = idx_block + b * N
    cnt_tile = jnp.sum((work == inf).astype(jnp.float32), axis=0)

    @pl.when(t == 0)
    def _():
        cnt_ref[0, 0, :] = jnp.zeros((N,), jnp.float32)

    cnt_ref[0, 0, :] += cnt_tile


def _run_knn(nxr, xyz):
    return pl.pallas_call(
        _knn_body,
        grid=(B, NT_ROWS),
        in_specs=[
            pl.BlockSpec((1, MT, 3), lambda b, t: (b, t, 0)),
            pl.BlockSpec((1, 3, N), lambda b, t: (b, 0, 0)),
        ],
        out_specs=[
            pl.BlockSpec((1, MT, K), lambda b, t: (b, t, 0)),
            pl.BlockSpec((1, 1, N), lambda b, t: (b, 0, 0)),
        ],
        out_shape=[
            jax.ShapeDtypeStruct((B, NP, K), jnp.int32),
            jax.ShapeDtypeStruct((B, 1, N), jnp.float32),
        ],
    )(nxr, xyz)


def _mlp_body(p_ref, cnt_ref, w1t_ref, b1_ref, g1_ref, be1_ref,
              w2t_ref, b2_ref, g2_ref, be2_ref, out_ref, h1s_ref, h2s_ref):
    ntot = jnp.float32(B * NP * K)
    hi = jax.lax.Precision.HIGHEST
    s1 = jnp.zeros((1, C1), jnp.float32)
    q1 = jnp.zeros((1, C1), jnp.float32)
    for b in range(B):
        h = jnp.dot(p_ref[b], w1t_ref[...],
                    preferred_element_type=jnp.float32) + b1_ref[...]
        h1s_ref[b] = h
        c = cnt_ref[b]
        s1 += jnp.dot(c, h, preferred_element_type=jnp.float32, precision=hi)
        q1 += jnp.dot(c, h * h, preferred_element_type=jnp.float32,
                      precision=hi)
    m1 = s1 / ntot
    v1 = q1 / ntot - m1 * m1
    sc1 = g1_ref[...] / jnp.sqrt(v1 + 1e-5)
    sh1 = be1_ref[...] - m1 * sc1

    s2 = jnp.zeros((1, C2), jnp.float32)
    q2 = jnp.zeros((1, C2), jnp.float32)
    for b in range(B):
        h1 = jnp.maximum(h1s_ref[b] * sc1 + sh1, 0.0)
        h2 = jnp.dot(h1, w2t_ref[...],
                     preferred_element_type=jnp.float32) + b2_ref[...]
        h2s_ref[b] = h2
        c = cnt_ref[b]
        s2 += jnp.dot(c, h2, preferred_element_type=jnp.float32, precision=hi)
        q2 += jnp.dot(c, h2 * h2, preferred_element_type=jnp.float32,
                      precision=hi)
    m2 = s2 / ntot
    v2 = q2 / ntot - m2 * m2
    sc2 = g2_ref[...] / jnp.sqrt(v2 + 1e-5)
    sh2 = be2_ref[...] - m2 * sc2
    for b in range(B):
        out_ref[b] = jnp.maximum(h2s_ref[b] * sc2 + sh2, 0.0)


def _run_mlp(p_pm, counts, W1, b1, gamma1, beta1, W2, b2, gamma2, beta2):
    shapes = [(B, N, C1), (B, 1, N), (C1, C1), (1, C1), (1, C1), (1, C1),
              (C1, C2), (1, C2), (1, C2), (1, C2)]
    return pl.pallas_call(
        _mlp_body,
        grid=(1,),
        in_specs=[pl.BlockSpec(s, lambda _, _s=s: tuple(0 for _ in _s))
                  for s in shapes],
        out_specs=pl.BlockSpec((B, N, C2), lambda _: (0, 0, 0)),
        out_shape=jax.ShapeDtypeStruct((B, N, C2), jnp.float32),
        scratch_shapes=[
            pltpu.VMEM((B, N, C1), jnp.float32),
            pltpu.VMEM((B, N, C2), jnp.float32),
        ],
    )(p_pm, counts, W1.T, b1.reshape(1, C1), gamma1.reshape(1, C1),
      beta1.reshape(1, C1), W2.T, b2.reshape(1, C2), gamma2.reshape(1, C2),
      beta2.reshape(1, C2))


NWORK = 32
RPW = (B * NP) // NWORK
G = 4


def _pool_body(h2_hbm, idx_hbm, out_hbm, idx_v, bufs, out_v, sem):
    wid = lax.axis_index("s") * 2 + lax.axis_index("c")
    base = wid * RPW
    pltpu.sync_copy(idx_hbm.at[pl.ds(base, RPW)], idx_v)

    def chunk(ci, _):
        r0 = ci * G
        handles = []
        for j in range(G):
            handles.append(pltpu.async_copy(
                h2_hbm.at[idx_v.at[r0 + j]], bufs.at[j], sem))
        for h in handles:
            h.wait()
        for j in range(G):
            for c in range(C2 // 16):
                acc = bufs[j, 0, pl.ds(c * 16, 16)]
                for r in range(1, K):
                    acc = jnp.maximum(acc, bufs[j, r, pl.ds(c * 16, 16)])
                out_v[r0 + j, pl.ds(c * 16, 16)] = acc
        return 0

    lax.fori_loop(0, RPW // G, chunk, 0)
    pltpu.sync_copy(out_v, out_hbm.at[pl.ds(base, RPW)])


@functools.cache
def _make_pool_kernel():
    return pl.kernel(
        _pool_body,
        mesh=plsc.VectorSubcoreMesh(core_axis_name="c", subcore_axis_name="s"),
        out_type=jax.ShapeDtypeStruct((B * NP, C2), jnp.float32),
        scratch_types=[
            pltpu.VMEM((RPW, K), jnp.int32),
            pltpu.VMEM((G, K, C2), jnp.float32),
            pltpu.VMEM((RPW, C2), jnp.float32),
            pltpu.SemaphoreType.DMA,
        ],
    )


def kernel(xyz, points, W1, b1, gamma1, beta1, W2, b2, gamma2, beta2):
    nxr = _run_fps(xyz)
    idxg, counts = _run_knn(nxr, xyz)
    p_pm = jnp.transpose(points, (0, 2, 1))
    h2 = _run_mlp(p_pm, counts, W1, b1, gamma1, beta1,
                  W2, b2, gamma2, beta2)
    pooled = _make_pool_kernel()(h2.reshape(B * N, C2),
                                 idxg.reshape(B * NP, K))
    new_xyz = jnp.transpose(nxr, (0, 2, 1))
    return new_xyz, jnp.transpose(pooled.reshape(B, NP, C2), (0, 2, 1))
